# baseline (speedup 1.0000x reference)
import functools

import jax
import jax.numpy as jnp
import numpy as np
from jax import lax
from jax.experimental import pallas as pl
from jax.experimental.pallas import tpu as pltpu
from jax.experimental.pallas import tpu_sc as plsc

TBL = 1000
TBL_PAD = 1024
IN_DIM = 256
D = 1024
B = 16384

NC = 2
NS = 16
NW = NC * NS
B_PER_W = B // NW
CHUNK = 32
NCH = B_PER_W // CHUNK

_PERM = np.arange(D).reshape(D // 32, 2, 16).transpose(0, 2, 1).reshape(D)


def _sigmoid(x):
    return 1.0 / (1.0 + jnp.exp(-x))


def _mlp_table_body(e_ref, w1_ref, b1_ref, w2_ref, b2_ref, y_ref):
    e = jnp.concatenate(
        [e_ref[...], jnp.zeros((TBL_PAD - TBL, IN_DIM), jnp.float32)], axis=0)
    h = jnp.dot(e, w1_ref[...], preferred_element_type=jnp.float32)
    h = h + b1_ref[...]
    h = h * _sigmoid(h)
    y = jnp.dot(h, w2_ref[...], preferred_element_type=jnp.float32)
    y = y + b2_ref[...]
    y_ref[...] = (y * _sigmoid(y)).astype(jnp.bfloat16)


def _mlp_table(e, W1, b1, W2, b2):
    return pl.pallas_call(
        _mlp_table_body,
        out_shape=jax.ShapeDtypeStruct((TBL_PAD, D), jnp.bfloat16),
    )(e, W1, b1.reshape(1, D), W2, b2.reshape(1, D))


_sc_mesh = plsc.VectorSubcoreMesh(core_axis_name="c", subcore_axis_name="s")


@functools.partial(
    pl.kernel,
    out_type=jax.ShapeDtypeStruct((B, D), jnp.int32),
    mesh=_sc_mesh,
    scratch_types=[
        pltpu.VMEM((NCH, CHUNK), jnp.int32),
        pltpu.VMEM((CHUNK, D // 2), jnp.int32),
        pltpu.VMEM((CHUNK, D // 2), jnp.int32),
        pltpu.VMEM((CHUNK, D), jnp.int32),
        pltpu.VMEM((CHUNK, D), jnp.int32),
        pltpu.SemaphoreType.DMA,
        pltpu.SemaphoreType.DMA,
        pltpu.SemaphoreType.DMA,
        pltpu.SemaphoreType.DMA,
    ],
)
def _sc_gather(table_hbm, idx_hbm, out_hbm, idx_v, bbuf0, bbuf1, fbuf0, fbuf1,
               g0, g1, p0, p1):
    wid = lax.axis_index("s") * NC + lax.axis_index("c")
    base = wid * B_PER_W
    bbufs = (bbuf0, bbuf1)
    fbufs = (fbuf0, fbuf1)
    gsem = (g0, g1)
    psem = (p0, p1)
    pltpu.sync_copy(idx_hbm.at[wid], idx_v)
    gets = [None, None]
    puts = [None, None]
    gets[0] = pltpu.async_copy(table_hbm.at[idx_v.at[0]], bbufs[0], gsem[0])
    for c in range(NCH):
        cur = c % 2
        nxt = (c + 1) % 2
        gets[cur].wait()
        if c + 1 < NCH:
            gets[nxt] = pltpu.async_copy(
                table_hbm.at[idx_v.at[c + 1]], bbufs[nxt], gsem[nxt])
        if puts[cur] is not None:
            puts[cur].wait()
        bbuf = bbufs[cur]
        fbuf = fbufs[cur]

        def upcast_row(r, _, bbuf=bbuf, fbuf=fbuf):
            for k in range(D // 32):
                x = bbuf[r, pl.ds(16 * k, 16)]
                fbuf[r, pl.ds(32 * k, 16)] = x << 16
                fbuf[r, pl.ds(32 * k + 16, 16)] = x & jnp.int32(-65536)
            return 0

        lax.fori_loop(0, CHUNK, upcast_row, 0)
        puts[cur] = pltpu.async_copy(
            fbuf, out_hbm.at[pl.ds(base + c * CHUNK, CHUNK)], psem[cur])
    puts[(NCH - 2) % 2].wait()
    puts[(NCH - 1) % 2].wait()


def kernel(diffusion_step, embedding, W1, b1, W2, b2):
    y = _mlp_table(embedding, W1, b1, W2, b2)
    y_perm = y[:, _PERM]
    y_i32 = lax.bitcast_convert_type(
        y_perm.reshape(TBL_PAD, D // 2, 2), jnp.int32)
    idx = diffusion_step.reshape(NW, NCH, CHUNK)
    out = _sc_gather(y_i32, idx)
    return lax.bitcast_convert_type(out, jnp.float32)


# speedup vs baseline: 1.3095x; 1.3095x over previous
import functools

import jax
import jax.numpy as jnp
import numpy as np
from jax import lax
from jax.experimental import pallas as pl
from jax.experimental.pallas import tpu as pltpu
from jax.experimental.pallas import tpu_sc as plsc

TBL = 1000
TBL_PAD = 1024
IN_DIM = 256
D = 1024
B = 16384

NC = 2
NS = 16
NW = NC * NS
B_PER_W = B // NW
CHUNK = 32
NCH = B_PER_W // CHUNK

_PERM = np.arange(D).reshape(D // 32, 2, 16).transpose(0, 2, 1).reshape(D)


def _sigmoid(x):
    return 1.0 / (1.0 + jnp.exp(-x))


def _mlp_table_body(e_ref, w1_ref, b1_ref, w2_ref, b2_ref, y_ref):
    e = jnp.concatenate(
        [e_ref[...], jnp.zeros((TBL_PAD - TBL, IN_DIM), jnp.float32)], axis=0)
    h = jnp.dot(e, w1_ref[...], preferred_element_type=jnp.float32)
    h = h + b1_ref[...]
    h = h * _sigmoid(h)
    y = jnp.dot(h, w2_ref[...], preferred_element_type=jnp.float32)
    y = y + b2_ref[...]
    y_ref[...] = (y * _sigmoid(y)).astype(jnp.bfloat16)


def _mlp_table(e, W1, b1, W2, b2):
    return pl.pallas_call(
        _mlp_table_body,
        out_shape=jax.ShapeDtypeStruct((TBL_PAD, D), jnp.bfloat16),
    )(e, W1, b1.reshape(1, D), W2, b2.reshape(1, D))


_sc_mesh = plsc.VectorSubcoreMesh(core_axis_name="c", subcore_axis_name="s")


@functools.partial(
    pl.kernel,
    out_type=jax.ShapeDtypeStruct((B, D), jnp.int32),
    mesh=_sc_mesh,
    scratch_types=[
        pltpu.VMEM((NCH, CHUNK), jnp.int32),
        pltpu.VMEM((CHUNK, D // 2), jnp.int32),
        pltpu.VMEM((CHUNK, D // 2), jnp.int32),
        pltpu.VMEM((CHUNK, D), jnp.int32),
        pltpu.VMEM((CHUNK, D), jnp.int32),
        pltpu.SemaphoreType.DMA,
        pltpu.SemaphoreType.DMA,
        pltpu.SemaphoreType.DMA,
        pltpu.SemaphoreType.DMA,
    ],
)
def _sc_gather(table_hbm, idx_hbm, out_hbm, idx_v, bbuf0, bbuf1, fbuf0, fbuf1,
               g0, g1, p0, p1):
    wid = lax.axis_index("s") * NC + lax.axis_index("c")
    base = wid * B_PER_W
    bbufs = (bbuf0, bbuf1)
    fbufs = (fbuf0, fbuf1)
    gsem = (g0, g1)
    psem = (p0, p1)
    pltpu.sync_copy(idx_hbm.at[wid], idx_v)
    gets = [None, None]
    puts = [None, None]
    gets[0] = pltpu.async_copy(table_hbm.at[idx_v.at[0]], bbufs[0], gsem[0])
    for c in range(NCH):
        cur = c % 2
        nxt = (c + 1) % 2
        gets[cur].wait()
        if c + 1 < NCH:
            gets[nxt] = pltpu.async_copy(
                table_hbm.at[idx_v.at[c + 1]], bbufs[nxt], gsem[nxt])
        if puts[cur] is not None:
            puts[cur].wait()
        bbuf = bbufs[cur]
        fbuf = fbufs[cur]

        @plsc.parallel_loop(0, CHUNK * (D // 32), 1, unroll=4)
        def upcast_g(g, bbuf=bbuf, fbuf=fbuf):
            r = g >> 5
            k = g & 31
            x = bbuf[r, pl.ds(16 * k, 16)]
            fbuf[r, pl.ds(32 * k, 16)] = x << 16
            fbuf[r, pl.ds(32 * k + 16, 16)] = x & jnp.int32(-65536)
        puts[cur] = pltpu.async_copy(
            fbuf, out_hbm.at[pl.ds(base + c * CHUNK, CHUNK)], psem[cur])
    puts[(NCH - 2) % 2].wait()
    puts[(NCH - 1) % 2].wait()


def kernel(diffusion_step, embedding, W1, b1, W2, b2):
    y = _mlp_table(embedding, W1, b1, W2, b2)
    y_perm = y[:, _PERM]
    y_i32 = lax.bitcast_convert_type(
        y_perm.reshape(TBL_PAD, D // 2, 2), jnp.int32)
    idx = diffusion_step.reshape(NW, NCH, CHUNK)
    out = _sc_gather(y_i32, idx)
    return lax.bitcast_convert_type(out, jnp.float32)


# speedup vs baseline: 2.2149x; 1.6914x over previous
import functools

import jax
import jax.numpy as jnp
from jax import lax
from jax.experimental import pallas as pl
from jax.experimental.pallas import tpu as pltpu
from jax.experimental.pallas import tpu_sc as plsc

TBL = 1000
TBL_PAD = 1024
IN_DIM = 256
D = 1024
B = 16384

NC = 2
NS = 16
NW = NC * NS
B_PER_W = B // NW
CHUNK = 32
NCH = B_PER_W // CHUNK


def _sigmoid(x):
    return 1.0 / (1.0 + jnp.exp(-x))


def _mlp_table_body(e_ref, w1_ref, b1_ref, w2_ref, b2_ref, y_ref):
    e = jnp.concatenate(
        [e_ref[...], jnp.zeros((TBL_PAD - TBL, IN_DIM), jnp.float32)], axis=0)
    h = jnp.dot(e, w1_ref[...], preferred_element_type=jnp.float32)
    h = h + b1_ref[...]
    h = h * _sigmoid(h)
    y = jnp.dot(h, w2_ref[...], preferred_element_type=jnp.float32)
    y = y + b2_ref[...]
    y_ref[...] = y * _sigmoid(y)


def _mlp_table(e, W1, b1, W2, b2):
    return pl.pallas_call(
        _mlp_table_body,
        out_shape=jax.ShapeDtypeStruct((TBL_PAD, D), jnp.float32),
    )(e, W1, b1.reshape(1, D), W2, b2.reshape(1, D))


_sc_mesh = plsc.VectorSubcoreMesh(core_axis_name="c", subcore_axis_name="s")


@functools.partial(
    pl.kernel,
    out_type=jax.ShapeDtypeStruct((B, D), jnp.float32),
    mesh=_sc_mesh,
    scratch_types=[
        pltpu.VMEM((NCH, CHUNK), jnp.int32),
        pltpu.VMEM((CHUNK, D), jnp.float32),
        pltpu.VMEM((CHUNK, D), jnp.float32),
        pltpu.SemaphoreType.DMA,
        pltpu.SemaphoreType.DMA,
        pltpu.SemaphoreType.DMA,
        pltpu.SemaphoreType.DMA,
    ],
)
def _sc_gather(table_hbm, idx_hbm, out_hbm, idx_v, buf0, buf1, g0, g1, p0, p1):
    wid = lax.axis_index("s") * NC + lax.axis_index("c")
    base = wid * B_PER_W
    bufs = (buf0, buf1)
    gsem = (g0, g1)
    psem = (p0, p1)
    pltpu.sync_copy(idx_hbm.at[wid], idx_v)
    gets = [None, None]
    puts = [None, None]
    gets[0] = pltpu.async_copy(table_hbm.at[idx_v.at[0]], bufs[0], gsem[0])
    for c in range(NCH):
        cur = c % 2
        nxt = (c + 1) % 2
        gets[cur].wait()
        if c + 1 < NCH:
            if puts[nxt] is not None:
                puts[nxt].wait()
            gets[nxt] = pltpu.async_copy(
                table_hbm.at[idx_v.at[c + 1]], bufs[nxt], gsem[nxt])
        puts[cur] = pltpu.async_copy(
            bufs[cur], out_hbm.at[pl.ds(base + c * CHUNK, CHUNK)], psem[cur])
    puts[(NCH - 2) % 2].wait()
    puts[(NCH - 1) % 2].wait()


def kernel(diffusion_step, embedding, W1, b1, W2, b2):
    y = _mlp_table(embedding, W1, b1, W2, b2)
    idx = diffusion_step.reshape(NW, NCH, CHUNK)
    return _sc_gather(y, idx)


# speedup vs baseline: 2.2899x; 1.0339x over previous
import functools

import jax
import jax.numpy as jnp
from jax import lax
from jax.experimental import pallas as pl
from jax.experimental.pallas import tpu as pltpu
from jax.experimental.pallas import tpu_sc as plsc

TBL = 1000
TBL_PAD = 1024
IN_DIM = 256
D = 1024
B = 16384

NC = 2
NS = 16
NW = NC * NS
B_PER_W = B // NW
CHUNK = 16
NCH = B_PER_W // CHUNK
NBUF = 4


def _sigmoid(x):
    return 1.0 / (1.0 + jnp.exp(-x))


def _mlp_table_body(e_ref, w1_ref, b1_ref, w2_ref, b2_ref, y_ref):
    e = jnp.concatenate(
        [e_ref[...], jnp.zeros((TBL_PAD - TBL, IN_DIM), jnp.float32)], axis=0)
    h = jnp.dot(e, w1_ref[...], preferred_element_type=jnp.float32)
    h = h + b1_ref[...]
    h = h * _sigmoid(h)
    y = jnp.dot(h, w2_ref[...], preferred_element_type=jnp.float32)
    y = y + b2_ref[...]
    y_ref[...] = y * _sigmoid(y)


def _mlp_table(e, W1, b1, W2, b2):
    return pl.pallas_call(
        _mlp_table_body,
        out_shape=jax.ShapeDtypeStruct((TBL_PAD, D), jnp.float32),
    )(e, W1, b1.reshape(1, D), W2, b2.reshape(1, D))


_sc_mesh = plsc.VectorSubcoreMesh(core_axis_name="c", subcore_axis_name="s")


@functools.partial(
    pl.kernel,
    out_type=jax.ShapeDtypeStruct((B, D), jnp.float32),
    mesh=_sc_mesh,
    scratch_types=(
        [pltpu.VMEM((NCH, CHUNK), jnp.int32)]
        + [pltpu.VMEM((CHUNK, D), jnp.float32)] * NBUF
        + [pltpu.SemaphoreType.DMA] * (2 * NBUF)
    ),
)
def _sc_gather(table_hbm, idx_hbm, out_hbm, idx_v, *bufs_and_sems):
    bufs = bufs_and_sems[:NBUF]
    gsem = bufs_and_sems[NBUF:2 * NBUF]
    psem = bufs_and_sems[2 * NBUF:]
    wid = lax.axis_index("s") * NC + lax.axis_index("c")
    base = wid * B_PER_W
    pltpu.sync_copy(idx_hbm.at[wid], idx_v)
    gets = [None] * NBUF
    puts = [None] * NBUF

    def wait_put(slot):
        if puts[slot] is not None:
            puts[slot].wait()
            puts[slot] = None

    for c in range(min(2, NCH)):
        gets[c] = pltpu.async_copy(
            table_hbm.at[idx_v.at[c]], bufs[c], gsem[c])
    for c in range(NCH):
        b = c % NBUF
        gets[b].wait()
        c2 = c + 2
        if c2 < NCH:
            b2 = c2 % NBUF
            wait_put(b2)
            gets[b2] = pltpu.async_copy(
                table_hbm.at[idx_v.at[c2]], bufs[b2], gsem[b2])
        puts[b] = pltpu.async_copy(
            bufs[b], out_hbm.at[pl.ds(base + c * CHUNK, CHUNK)], psem[b])
    for b in range(NBUF):
        wait_put(b)


def kernel(diffusion_step, embedding, W1, b1, W2, b2):
    y = _mlp_table(embedding, W1, b1, W2, b2)
    idx = diffusion_step.reshape(NW, NCH, CHUNK)
    return _sc_gather(y, idx)


# speedup vs baseline: 2.3067x; 1.0073x over previous
import functools

import jax
import jax.numpy as jnp
from jax import lax
from jax.experimental import pallas as pl
from jax.experimental.pallas import tpu as pltpu
from jax.experimental.pallas import tpu_sc as plsc

TBL = 1000
TBL_PAD = 1024
IN_DIM = 256
D = 1024
B = 16384

NC = 2
NS = 16
NW = NC * NS
B_PER_W = B // NW
CHUNK = 16
NCH = B_PER_W // CHUNK
NBUF = 6


def _sigmoid(x):
    return 1.0 / (1.0 + jnp.exp(-x))


def _mlp_table_body(e_ref, w1_ref, b1_ref, w2_ref, b2_ref, y_ref):
    e = jnp.concatenate(
        [e_ref[...], jnp.zeros((TBL_PAD - TBL, IN_DIM), jnp.float32)], axis=0)
    h = jnp.dot(e, w1_ref[...], preferred_element_type=jnp.float32)
    h = h + b1_ref[...]
    h = h * _sigmoid(h)
    y = jnp.dot(h, w2_ref[...], preferred_element_type=jnp.float32)
    y = y + b2_ref[...]
    y_ref[...] = y * _sigmoid(y)


def _mlp_table(e, W1, b1, W2, b2):
    return pl.pallas_call(
        _mlp_table_body,
        out_shape=jax.ShapeDtypeStruct((TBL_PAD, D), jnp.float32),
    )(e, W1, b1.reshape(1, D), W2, b2.reshape(1, D))


_sc_mesh = plsc.VectorSubcoreMesh(core_axis_name="c", subcore_axis_name="s")


@functools.partial(
    pl.kernel,
    out_type=jax.ShapeDtypeStruct((B, D), jnp.float32),
    mesh=_sc_mesh,
    scratch_types=(
        [pltpu.VMEM((NCH, CHUNK), jnp.int32)]
        + [pltpu.VMEM((CHUNK, D), jnp.float32)] * NBUF
        + [pltpu.SemaphoreType.DMA] * (2 * NBUF)
    ),
)
def _sc_gather(table_hbm, idx_hbm, out_hbm, idx_v, *bufs_and_sems):
    bufs = bufs_and_sems[:NBUF]
    gsem = bufs_and_sems[NBUF:2 * NBUF]
    psem = bufs_and_sems[2 * NBUF:]
    wid = lax.axis_index("s") * NC + lax.axis_index("c")
    base = wid * B_PER_W
    pltpu.sync_copy(idx_hbm.at[wid], idx_v)
    gets = [None] * NBUF
    puts = [None] * NBUF

    def wait_put(slot):
        if puts[slot] is not None:
            puts[slot].wait()
            puts[slot] = None

    for c in range(min(3, NCH)):
        gets[c] = pltpu.async_copy(
            table_hbm.at[idx_v.at[c]], bufs[c], gsem[c])
    for c in range(NCH):
        b = c % NBUF
        gets[b].wait()
        c2 = c + 3
        if c2 < NCH:
            b2 = c2 % NBUF
            wait_put(b2)
            gets[b2] = pltpu.async_copy(
                table_hbm.at[idx_v.at[c2]], bufs[b2], gsem[b2])
        puts[b] = pltpu.async_copy(
            bufs[b], out_hbm.at[pl.ds(base + c * CHUNK, CHUNK)], psem[b])
    for b in range(NBUF):
        wait_put(b)


def kernel(diffusion_step, embedding, W1, b1, W2, b2):
    y = _mlp_table(embedding, W1, b1, W2, b2)
    idx = diffusion_step.reshape(NW, NCH, CHUNK)
    return _sc_gather(y, idx)
